# Initial kernel scaffold; baseline (speedup 1.0000x reference)
import jax
import jax.numpy as jnp
from jax.experimental import pallas as pl


def kernel(feat, embed, W_ih, W_hh, b_ih, b_hh, proj_W, proj_b):
    raise NotImplementedError("write your pallas kernel here")


# speedup vs baseline: 12.0130x; 12.0130x over previous
import jax
import jax.numpy as jnp
from jax.experimental import pallas as pl
from jax.experimental.pallas import tpu as pltpu

NCHAR = 113
TLEN = 201
HDIM = 512
BATCH = 8
CPAD = 128


def _decode_body(feat_ref, sos_ref, wih_ref, whh_ref, bih_ref, bhh_ref,
                 pw_ref, pb_ref, out_ref, xp_scr, seq_scr):
    wih = wih_ref[...]
    whh = whh_ref[...]
    bih = bih_ref[...]
    bhh = bhh_ref[...]

    sos = jnp.broadcast_to(sos_ref[...], (BATCH, HDIM))
    xp0 = jnp.dot(sos_ref[...], wih, preferred_element_type=jnp.float32) + bih
    xp_scr[0] = jnp.broadcast_to(xp0, (BATCH, 3 * HDIM))
    seq_scr[0:BATCH, :] = sos

    def inner(k, h):
        xp = xp_scr[k]
        hp = jnp.dot(h, whh, preferred_element_type=jnp.float32) + bhh
        rz = jax.nn.sigmoid(xp[:, :2 * HDIM] + hp[:, :2 * HDIM])
        r = rz[:, :HDIM]
        z = rz[:, HDIM:]
        n = jnp.tanh(xp[:, 2 * HDIM:] + r * hp[:, 2 * HDIM:])
        return (1.0 - z) * n + z * h

    def outer(t, h):
        h_fin = jax.lax.fori_loop(0, t + 1, inner, h)
        xp_scr[t + 1] = (jnp.dot(h_fin, wih, preferred_element_type=jnp.float32)
                         + bih)
        seq_scr[pl.ds((t + 1) * BATCH, BATCH), :] = h_fin
        return h_fin

    jax.lax.fori_loop(0, TLEN - 1, outer, feat_ref[...])

    seq = seq_scr[...]
    out_ref[...] = (jnp.dot(seq, pw_ref[...], preferred_element_type=jnp.float32)
                    + pb_ref[...])


def kernel(feat, embed, W_ih, W_hh, b_ih, b_hh, proj_W, proj_b):
    sos = embed[0:1, :]
    wih_t = W_ih.T
    whh_t = W_hh.T
    bih2 = b_ih.reshape(1, 3 * HDIM)
    bhh2 = b_hh.reshape(1, 3 * HDIM)
    pw_t = jnp.zeros((HDIM, CPAD), jnp.float32).at[:, :NCHAR].set(proj_W.T)
    pb2 = jnp.zeros((1, CPAD), jnp.float32).at[:, :NCHAR].set(proj_b)

    out = pl.pallas_call(
        _decode_body,
        out_shape=jax.ShapeDtypeStruct((TLEN * BATCH, CPAD), jnp.float32),
        scratch_shapes=[
            pltpu.VMEM((TLEN, BATCH, 3 * HDIM), jnp.float32),
            pltpu.VMEM((TLEN * BATCH, HDIM), jnp.float32),
        ],
    )(feat, sos, wih_t, whh_t, bih2, bhh2, pw_t, pb2)

    seq_bc = out.reshape(TLEN, BATCH, CPAD)[:, :, :NCHAR]
    return seq_bc.transpose(1, 2, 0)
